# Initial kernel scaffold; baseline (speedup 1.0000x reference)
import jax
import jax.numpy as jnp
from jax.experimental import pallas as pl


def kernel(x, tables):
    raise NotImplementedError("write your pallas kernel here")


# speedup vs baseline: 3.9482x; 3.9482x over previous
import functools

import jax
import jax.numpy as jnp
from jax import lax
from jax.experimental import pallas as pl
from jax.experimental.pallas import tpu as pltpu
from jax.experimental.pallas import tpu_sc as plsc

NFEAT = 128
MAX_SIZE = 100
NUM_STAMPS = 8
LANES = 16

NUM_CORES = 2
NUM_SUBCORES = 16
NUM_WORKERS = NUM_CORES * NUM_SUBCORES

ROWS_PER_CHUNK = 16
GATHER_ROWS = ROWS_PER_CHUNK * NUM_STAMPS


def _sc_kernel(n_rows, xflat_hbm, table_hbm, out_hbm, idx_v, rows_v, acc_v, gsem):
    rows_per_worker = n_rows // NUM_WORKERS
    chunks = rows_per_worker // ROWS_PER_CHUNK
    wid = lax.axis_index("s") * NUM_CORES + lax.axis_index("c")
    base_row = wid * rows_per_worker

    pat = (lax.iota(jnp.int32, LANES) % NUM_STAMPS) * MAX_SIZE

    @pl.loop(0, chunks)
    def _(c):
        r0 = base_row + c * ROWS_PER_CHUNK
        pltpu.sync_copy(xflat_hbm.at[pl.ds(r0 * NUM_STAMPS, GATHER_ROWS)], idx_v)

        for j in range(GATHER_ROWS // LANES):
            sl = pl.ds(j * LANES, LANES)
            idx_v[sl] = idx_v[sl] + pat

        pltpu.async_copy(table_hbm.at[idx_v], rows_v, gsem).wait()

        @pl.loop(0, ROWS_PER_CHUNK)
        def _(n):
            for f in range(NFEAT // LANES):
                sl = pl.ds(f * LANES, LANES)
                s = rows_v[n * NUM_STAMPS, sl]
                for i in range(1, NUM_STAMPS):
                    s = s + rows_v[n * NUM_STAMPS + i, sl]
                acc_v[n, sl] = s

        pltpu.sync_copy(acc_v, out_hbm.at[pl.ds(r0, ROWS_PER_CHUNK)])


def kernel(x, tables):
    b, l, num_stamps = x.shape
    n_rows = b * l
    xflat = jnp.asarray(x, jnp.int32).reshape(n_rows * num_stamps)
    tab2d = tables.reshape(tables.shape[0] * tables.shape[1], tables.shape[2])

    mesh = plsc.VectorSubcoreMesh(core_axis_name="c", subcore_axis_name="s")
    run = pl.kernel(
        functools.partial(_sc_kernel, n_rows),
        out_type=jax.ShapeDtypeStruct((n_rows, NFEAT), tables.dtype),
        mesh=mesh,
        scratch_types=[
            pltpu.VMEM((GATHER_ROWS,), jnp.int32),
            pltpu.VMEM((GATHER_ROWS, NFEAT), jnp.float32),
            pltpu.VMEM((ROWS_PER_CHUNK, NFEAT), jnp.float32),
            pltpu.SemaphoreType.DMA,
        ],
    )
    out = run(xflat, tab2d)
    return out.reshape(b, l, NFEAT)


# speedup vs baseline: 6.8122x; 1.7254x over previous
import functools

import jax
import jax.numpy as jnp
from jax import lax
from jax.experimental import pallas as pl
from jax.experimental.pallas import tpu as pltpu
from jax.experimental.pallas import tpu_sc as plsc

NFEAT = 128
MAX_SIZE = 100
NUM_STAMPS = 8
LANES = 16

NUM_CORES = 2
NUM_SUBCORES = 16
NUM_WORKERS = NUM_CORES * NUM_SUBCORES

ROWS_PER_CHUNK = 16
GATHER_ROWS = ROWS_PER_CHUNK * NUM_STAMPS
NBUF = 2


def _sc_kernel(n_rows, x2d_hbm, table_hbm, out_hbm,
               fidx_v, rows_v, acc_v, gsems, osems):
    rows_per_worker = n_rows // NUM_WORKERS
    chunks = rows_per_worker // ROWS_PER_CHUNK
    wid = lax.axis_index("s") * NUM_CORES + lax.axis_index("c")
    base_row = wid * rows_per_worker

    pltpu.sync_copy(x2d_hbm.at[pl.ds(wid * chunks, chunks)], fidx_v)
    pat = (lax.iota(jnp.int32, LANES) % NUM_STAMPS) * MAX_SIZE

    @pl.loop(0, chunks)
    def _(r):
        for g in range(GATHER_ROWS // LANES):
            sl = pl.ds(g * LANES, LANES)
            fidx_v[r, sl] = fidx_v[r, sl] + pat

    def start_gather(ch, b):
        pltpu.async_copy(table_hbm.at[fidx_v.at[ch]], rows_v.at[b], gsems[b])

    def wait_gather(ch, b):
        pltpu.make_async_copy(table_hbm.at[fidx_v.at[ch]], rows_v.at[b],
                              gsems[b]).wait()

    def compute(b):
        @pl.loop(0, ROWS_PER_CHUNK)
        def _(n):
            for f in range(NFEAT // LANES):
                sl = pl.ds(f * LANES, LANES)
                s = rows_v[b, n * NUM_STAMPS, sl]
                for i in range(1, NUM_STAMPS):
                    s = s + rows_v[b, n * NUM_STAMPS + i, sl]
                acc_v[b, n, sl] = s

    def start_out(ch, b):
        r0 = base_row + ch * ROWS_PER_CHUNK
        pltpu.async_copy(acc_v.at[b], out_hbm.at[pl.ds(r0, ROWS_PER_CHUNK)],
                         osems[b])

    def wait_out(ch, b):
        r0 = base_row + ch * ROWS_PER_CHUNK
        pltpu.make_async_copy(acc_v.at[b],
                              out_hbm.at[pl.ds(r0, ROWS_PER_CHUNK)],
                              osems[b]).wait()

    for b in range(NBUF):
        start_gather(b, b)

    @pl.loop(0, chunks - NBUF, step=NBUF)
    def _(c):
        for b in range(NBUF):
            ch = c + b

            @pl.when(ch >= NBUF)
            def _():
                wait_out(ch - NBUF, b)

            wait_gather(ch, b)
            compute(b)
            start_out(ch, b)
            start_gather(ch + NBUF, b)

    for b in range(NBUF):
        ch = chunks - NBUF + b
        if ch >= NBUF:
            wait_out(ch - NBUF, b)
        wait_gather(ch, b)
        compute(b)
        start_out(ch, b)
    for b in range(NBUF):
        wait_out(chunks - NBUF + b, b)


def kernel(x, tables):
    b, l, num_stamps = x.shape
    n_rows = b * l
    x2d = jnp.asarray(x, jnp.int32).reshape(
        n_rows * num_stamps // GATHER_ROWS, GATHER_ROWS)
    tab2d = tables.reshape(tables.shape[0] * tables.shape[1], tables.shape[2])
    chunks = n_rows // NUM_WORKERS // ROWS_PER_CHUNK

    mesh = plsc.VectorSubcoreMesh(core_axis_name="c", subcore_axis_name="s")
    run = pl.kernel(
        functools.partial(_sc_kernel, n_rows),
        out_type=jax.ShapeDtypeStruct((n_rows, NFEAT), tables.dtype),
        mesh=mesh,
        scratch_types=[
            pltpu.VMEM((chunks, GATHER_ROWS), jnp.int32),
            pltpu.VMEM((NBUF, GATHER_ROWS, NFEAT), jnp.float32),
            pltpu.VMEM((NBUF, ROWS_PER_CHUNK, NFEAT), jnp.float32),
            [pltpu.SemaphoreType.DMA] * NBUF,
            [pltpu.SemaphoreType.DMA] * NBUF,
        ],
    )
    out = run(x2d, tab2d)
    return out.reshape(b, l, NFEAT)


# speedup vs baseline: 7.0704x; 1.0379x over previous
import functools

import jax
import jax.numpy as jnp
from jax import lax
from jax.experimental import pallas as pl
from jax.experimental.pallas import tpu as pltpu
from jax.experimental.pallas import tpu_sc as plsc

NFEAT = 128
MAX_SIZE = 100
NUM_STAMPS = 8
LANES = 16

NUM_CORES = 2
NUM_SUBCORES = 16
NUM_WORKERS = NUM_CORES * NUM_SUBCORES

ROWS_PER_CHUNK = 16
GATHER_ROWS = ROWS_PER_CHUNK * NUM_STAMPS
NBUF = 2


def _sc_kernel(n_rows, x2d_hbm, table_hbm, out_hbm,
               fidx_v, rows_v, acc_v, gsems, osems):
    rows_per_worker = n_rows // NUM_WORKERS
    chunks = rows_per_worker // ROWS_PER_CHUNK
    wid = lax.axis_index("s") * NUM_CORES + lax.axis_index("c")
    base_row = wid * rows_per_worker

    pltpu.sync_copy(x2d_hbm.at[pl.ds(wid * chunks, chunks)], fidx_v)
    pat = (lax.iota(jnp.int32, LANES) % NUM_STAMPS) * MAX_SIZE

    @pl.loop(0, chunks)
    def _(r):
        for g in range(GATHER_ROWS // LANES):
            sl = pl.ds(g * LANES, LANES)
            fidx_v[r, sl] = fidx_v[r, sl] + pat

    def start_gather(ch, b):
        pltpu.async_copy(table_hbm.at[fidx_v.at[ch]], rows_v.at[b], gsems[b])

    def wait_gather(ch, b):
        pltpu.make_async_copy(table_hbm.at[fidx_v.at[ch]], rows_v.at[b],
                              gsems[b]).wait()

    def compute(b):
        @plsc.parallel_loop(0, ROWS_PER_CHUNK, unroll=2)
        def _(n):
            for f in range(NFEAT // LANES):
                sl = pl.ds(f * LANES, LANES)
                v = [rows_v[b, n * NUM_STAMPS + i, sl] for i in range(NUM_STAMPS)]
                while len(v) > 1:
                    v = [v[i] + v[i + 1] for i in range(0, len(v), 2)]
                acc_v[b, n, sl] = v[0]

    def start_out(ch, b):
        r0 = base_row + ch * ROWS_PER_CHUNK
        pltpu.async_copy(acc_v.at[b], out_hbm.at[pl.ds(r0, ROWS_PER_CHUNK)],
                         osems[b])

    def wait_out(ch, b):
        r0 = base_row + ch * ROWS_PER_CHUNK
        pltpu.make_async_copy(acc_v.at[b],
                              out_hbm.at[pl.ds(r0, ROWS_PER_CHUNK)],
                              osems[b]).wait()

    for b in range(NBUF):
        start_gather(b, b)

    @pl.loop(0, chunks - NBUF, step=NBUF)
    def _(c):
        for b in range(NBUF):
            ch = c + b

            @pl.when(ch >= NBUF)
            def _():
                wait_out(ch - NBUF, b)

            wait_gather(ch, b)
            compute(b)
            start_out(ch, b)
            start_gather(ch + NBUF, b)

    for b in range(NBUF):
        ch = chunks - NBUF + b
        if ch >= NBUF:
            wait_out(ch - NBUF, b)
        wait_gather(ch, b)
        compute(b)
        start_out(ch, b)
    for b in range(NBUF):
        wait_out(chunks - NBUF + b, b)


def kernel(x, tables):
    b, l, num_stamps = x.shape
    n_rows = b * l
    x2d = jnp.asarray(x, jnp.int32).reshape(
        n_rows * num_stamps // GATHER_ROWS, GATHER_ROWS)
    tab2d = tables.reshape(tables.shape[0] * tables.shape[1], tables.shape[2])
    chunks = n_rows // NUM_WORKERS // ROWS_PER_CHUNK

    mesh = plsc.VectorSubcoreMesh(core_axis_name="c", subcore_axis_name="s")
    run = pl.kernel(
        functools.partial(_sc_kernel, n_rows),
        out_type=jax.ShapeDtypeStruct((n_rows, NFEAT), tables.dtype),
        mesh=mesh,
        scratch_types=[
            pltpu.VMEM((chunks, GATHER_ROWS), jnp.int32),
            pltpu.VMEM((NBUF, GATHER_ROWS, NFEAT), jnp.float32),
            pltpu.VMEM((NBUF, ROWS_PER_CHUNK, NFEAT), jnp.float32),
            [pltpu.SemaphoreType.DMA] * NBUF,
            [pltpu.SemaphoreType.DMA] * NBUF,
        ],
    )
    out = run(x2d, tab2d)
    return out.reshape(b, l, NFEAT)
